# baseline (speedup 1.0000x reference)
import jax
import jax.numpy as jnp
from jax.experimental import pallas as pl
from jax.experimental.pallas import tpu as pltpu

_BT = 512


def _fused_kernel(mat_o_ref, mat_r_ref, fo_ref, fr_ref,
                  tgt_o_ref, tgt_r_ref, w_o_ref, w_r_ref,
                  b_o_ref, b_r_ref, out_o_ref, out_r_ref,
                  srcq_o, srcq_r):
    @pl.when(pl.program_id(0) == 0)
    def _prep():
        srcq_o[...] = fo_ref[...].astype(jnp.bfloat16)
        srcq_r[...] = fr_ref[...].astype(jnp.bfloat16)

    def one(mat_ref, srcq, tgt_ref, w_ref, b_ref, out_ref):
        mat = mat_ref[...]
        m = mat > 0
        cnt = jnp.sum(m.astype(jnp.float32), axis=1, keepdims=True)
        acc = jnp.dot(m.astype(jnp.bfloat16), srcq[...],
                      preferred_element_type=jnp.float32)
        h = jnp.maximum(acc, 0.0)
        upd = jax.lax.dot_general(
            h, w_ref[...], (((1,), (1,)), ((), ())),
            preferred_element_type=jnp.float32)
        inv = jnp.where(cnt > 0, 1.0 / jnp.maximum(cnt, 1.0), 0.0)
        out_ref[...] = tgt_ref[...] + upd * inv + b_ref[...]

    one(mat_o_ref, srcq_r, tgt_o_ref, w_o_ref, b_o_ref, out_o_ref)
    one(mat_r_ref, srcq_o, tgt_r_ref, w_r_ref, b_r_ref, out_r_ref)


def kernel(feature_obj, feature_region, mat_object, mat_region,
           W_r2o, b_r2o, W_o2r, b_o2r):
    T, S = mat_object.shape
    D = feature_obj.shape[1]
    big = pl.BlockSpec((_BT, S), lambda i: (i, 0))
    ful = pl.BlockSpec((S, D), lambda i: (0, 0))
    row = pl.BlockSpec((_BT, D), lambda i: (i, 0))
    wsp = pl.BlockSpec((D, D), lambda i: (0, 0))
    bsp = pl.BlockSpec((1, D), lambda i: (0, 0))
    out_o, out_r = pl.pallas_call(
        _fused_kernel,
        grid=(T // _BT,),
        in_specs=[big, big, ful, ful, row, row, wsp, wsp, bsp, bsp],
        out_specs=[row, row],
        out_shape=[jax.ShapeDtypeStruct((T, D), jnp.float32),
                   jax.ShapeDtypeStruct((T, D), jnp.float32)],
        scratch_shapes=[pltpu.VMEM((S, D), jnp.bfloat16),
                        pltpu.VMEM((S, D), jnp.bfloat16)],
        compiler_params=pltpu.CompilerParams(
            dimension_semantics=("arbitrary",)),
    )(mat_object, mat_region, feature_obj, feature_region,
      feature_obj, feature_region, W_r2o, W_o2r,
      b_r2o.reshape(1, -1), b_o2r.reshape(1, -1))
    return (out_o, out_r)


# speedup vs baseline: 1.2004x; 1.2004x over previous
import jax
import jax.numpy as jnp
from jax.experimental import pallas as pl
from jax.experimental.pallas import tpu as pltpu

_BT = 256


def _fused_kernel(mat_o_ref, mat_r_ref, fo_ref, fr_ref,
                  w_o_ref, w_r_ref,
                  b_o_ref, b_r_ref, out_o_ref, out_r_ref,
                  srcq_o, srcq_r):
    @pl.when(pl.program_id(0) == 0)
    def _prep():
        srcq_o[...] = fo_ref[...].astype(jnp.bfloat16)
        srcq_r[...] = fr_ref[...].astype(jnp.bfloat16)

    row0 = pl.program_id(0) * _BT

    def one(mat_ref, srcq, f_ref, w_ref, b_ref, out_ref):
        mat = mat_ref[...]
        m = mat > 0
        cnt = jnp.sum(m.astype(jnp.float32), axis=1, keepdims=True)
        acc = jnp.dot(m.astype(jnp.bfloat16), srcq[...],
                      preferred_element_type=jnp.float32)
        h = jnp.maximum(acc, 0.0)
        upd = jax.lax.dot_general(
            h, w_ref[...], (((1,), (1,)), ((), ())),
            preferred_element_type=jnp.float32)
        inv = jnp.where(cnt > 0, 1.0 / jnp.maximum(cnt, 1.0), 0.0)
        tgt = f_ref[pl.ds(row0, _BT), :]
        out_ref[...] = tgt + upd * inv + b_ref[...]

    one(mat_o_ref, srcq_r, fo_ref, w_o_ref, b_o_ref, out_o_ref)
    one(mat_r_ref, srcq_o, fr_ref, w_r_ref, b_r_ref, out_r_ref)


def kernel(feature_obj, feature_region, mat_object, mat_region,
           W_r2o, b_r2o, W_o2r, b_o2r):
    T, S = mat_object.shape
    D = feature_obj.shape[1]
    big = pl.BlockSpec((_BT, S), lambda i: (i, 0))
    ful = pl.BlockSpec((S, D), lambda i: (0, 0))
    row = pl.BlockSpec((_BT, D), lambda i: (i, 0))
    wsp = pl.BlockSpec((D, D), lambda i: (0, 0))
    bsp = pl.BlockSpec((1, D), lambda i: (0, 0))
    out_o, out_r = pl.pallas_call(
        _fused_kernel,
        grid=(T // _BT,),
        in_specs=[big, big, ful, ful, wsp, wsp, bsp, bsp],
        out_specs=[row, row],
        out_shape=[jax.ShapeDtypeStruct((T, D), jnp.float32),
                   jax.ShapeDtypeStruct((T, D), jnp.float32)],
        scratch_shapes=[pltpu.VMEM((S, D), jnp.bfloat16),
                        pltpu.VMEM((S, D), jnp.bfloat16)],
        compiler_params=pltpu.CompilerParams(
            dimension_semantics=("arbitrary",)),
    )(mat_object, mat_region, feature_obj, feature_region,
      W_r2o, W_o2r, b_r2o.reshape(1, -1), b_o2r.reshape(1, -1))
    return (out_o, out_r)
